# baseline (speedup 1.0000x reference)
import functools

import jax
import jax.numpy as jnp
from jax import lax
from jax.experimental import pallas as pl
from jax.experimental.pallas import tpu as pltpu
from jax.experimental.pallas import tpu_sc as plsc

_T = 16 * 1024
_D = 256
_K = 8192

_TB = 512
_MM = 1024
_SUB = 256
_NT = _T // _TB
_WINDOW_END_SUBCHUNKS = (2816 // _SUB - 1, 5632 // _SUB - 1, _K // _SUB - 1)


def _argmin_body(x_ref, cb_ref, xn_ref, cn_ref, idx_ref):
    x_bf = x_ref[...].astype(jnp.bfloat16)
    xn = xn_ref[...]

    inf = jnp.full((_TB, 1), jnp.inf, jnp.float32)
    win_v, acc_v = inf, inf
    win_i = jnp.zeros((_TB, 1), jnp.int32)
    acc_i = jnp.zeros((_TB, 1), jnp.int32)
    iota = lax.broadcasted_iota(jnp.int32, (_TB, _SUB), 1)
    big = jnp.full((_TB, _SUB), jnp.int32(2**30))

    for mm in range(_K // _MM):
        cb_blk = cb_ref[pl.ds(mm * _MM, _MM), :].astype(jnp.bfloat16)
        s = lax.dot_general(
            x_bf, cb_blk, (((1,), (1,)), ((), ())),
            preferred_element_type=jnp.float32)
        cn_blk = cn_ref[0:1, pl.ds(mm * _MM, _MM)]
        dist_big = (xn + cn_blk) - 2.0 * s
        for j in range(_MM // _SUB):
            c = mm * (_MM // _SUB) + j
            d = dist_big[:, j * _SUB:(j + 1) * _SUB]
            m = jnp.min(d, axis=1, keepdims=True)
            io = jnp.min(jnp.where(d == m, iota, big),
                         axis=1, keepdims=True) + c * _SUB
            upd = m < win_v
            win_i = jnp.where(upd, io, win_i)
            win_v = jnp.where(upd, m, win_v)
            if c in _WINDOW_END_SUBCHUNKS:
                lt = win_v < acc_v
                acc_i = jnp.where(lt, win_i, acc_i)
                acc_v = jnp.where(lt, win_v, acc_v)
                acc_v = acc_v.astype(jnp.bfloat16).astype(jnp.float32)
                win_v, win_i = inf, jnp.zeros((_TB, 1), jnp.int32)

    idx_ref[...] = acc_i


def _argmin_call(x2d, cb, xn, cn):
    idx2 = pl.pallas_call(
        _argmin_body,
        grid=(_NT,),
        in_specs=[
            pl.BlockSpec((_TB, _D), lambda t: (t, 0)),
            pl.BlockSpec((_K, _D), lambda t: (0, 0)),
            pl.BlockSpec((_TB, 1), lambda t: (t, 0)),
            pl.BlockSpec((1, _K), lambda t: (0, 0)),
        ],
        out_specs=pl.BlockSpec((_TB, 1), lambda t: (t, 0)),
        out_shape=jax.ShapeDtypeStruct((_T, 1), jnp.int32),
        compiler_params=pltpu.CompilerParams(
            dimension_semantics=("arbitrary",)),
    )(x2d, cb, xn, cn)
    return idx2.reshape(_T)


_CH = 128


def _make_gather():
    info = plsc.get_sparse_core_info()
    nw = info.num_cores * info.num_subcores
    b_per_w = _T // nw
    n_chunks = b_per_w // _CH
    mesh = plsc.VectorSubcoreMesh(core_axis_name="c", subcore_axis_name="s")

    @functools.partial(
        pl.kernel, mesh=mesh,
        out_type=jax.ShapeDtypeStruct((_T, _D), jnp.float32),
        scratch_types=[
            pltpu.VMEM((_CH,), jnp.int32),
            pltpu.VMEM((_CH, _D), jnp.float32),
            pltpu.SemaphoreType.DMA,
        ],
    )
    def gather(idx_hbm, table_hbm, out_hbm, idx_v, rows_v, sem):
        wid = lax.axis_index("s") * info.num_cores + lax.axis_index("c")
        base = wid * b_per_w
        for c in range(n_chunks):
            off = base + c * _CH
            pltpu.sync_copy(idx_hbm.at[pl.ds(off, _CH)], idx_v)
            pltpu.async_copy(table_hbm.at[idx_v], rows_v, sem).wait()
            pltpu.sync_copy(rows_v, out_hbm.at[pl.ds(off, _CH)])

    return gather


def kernel(x, codebook):
    B, S, D = x.shape
    x2d = x.reshape(B * S, D)
    cb = codebook.reshape(-1, D)
    xn = jnp.sum(x ** 2, axis=-1, keepdims=True).reshape(B * S, 1)
    cn = jnp.sum(codebook[0] ** 2, axis=-1).reshape(1, -1)
    idx = _argmin_call(x2d, cb, xn, cn)
    out = _make_gather()(idx, cb)
    return out.reshape(B, S, D)


# speedup vs baseline: 1.2327x; 1.2327x over previous
import functools

import jax
import jax.numpy as jnp
from jax import lax
from jax.experimental import pallas as pl
from jax.experimental.pallas import tpu as pltpu
from jax.experimental.pallas import tpu_sc as plsc

_T = 16 * 1024
_D = 256
_K = 8192

_TB = 128
_MM = 1024
_LW = 128
_NT = _T // _TB
_WINDOW_END_STEPS = (2816 // _LW - 1, 5632 // _LW - 1, _K // _LW - 1)


def _argmin_body(x_ref, cb_ref, xn_ref, cn_ref, idx_ref):
    x_bf = x_ref[...].astype(jnp.bfloat16)
    xn = xn_ref[...]

    inf1 = jnp.full((_TB, 1), jnp.inf, jnp.float32)
    infl = jnp.full((_TB, _LW), jnp.inf, jnp.float32)
    zerl = jnp.zeros((_TB, _LW), jnp.float32)
    lane_iota = lax.broadcasted_iota(
        jnp.int32, (_TB, _LW), 1).astype(jnp.float32)
    lane_v, lane_s = infl, zerl
    acc_v, acc_i = inf1, jnp.zeros((_TB, 1), jnp.float32)

    for mm in range(_K // _MM):
        cb_blk = cb_ref[pl.ds(mm * _MM, _MM), :].astype(jnp.bfloat16)
        s = lax.dot_general(
            x_bf, cb_blk, (((1,), (1,)), ((), ())),
            preferred_element_type=jnp.float32)
        for j in range(_MM // _LW):
            gstep = mm * (_MM // _LW) + j
            cn_blk = cn_ref[0:1, pl.ds(gstep * _LW, _LW)]
            d = (xn + cn_blk) - 2.0 * s[:, j * _LW:(j + 1) * _LW]
            upd = d < lane_v
            lane_v = jnp.where(upd, d, lane_v)
            lane_s = jnp.where(upd, jnp.float32(gstep), lane_s)
            if gstep in _WINDOW_END_STEPS:
                m = jnp.min(lane_v, axis=1, keepdims=True)
                kk = lane_s * jnp.float32(_LW) + lane_iota
                io = jnp.min(jnp.where(lane_v == m, kk, jnp.float32(1e9)),
                             axis=1, keepdims=True)
                lt = m < acc_v
                acc_i = jnp.where(lt, io, acc_i)
                acc_v = jnp.where(lt, m, acc_v)
                acc_v = acc_v.astype(jnp.bfloat16).astype(jnp.float32)
                lane_v, lane_s = infl, zerl

    idx_ref[...] = acc_i.astype(jnp.int32)


def _argmin_call(x2d, cb, xn, cn):
    idx2 = pl.pallas_call(
        _argmin_body,
        grid=(_NT,),
        in_specs=[
            pl.BlockSpec((_TB, _D), lambda t: (t, 0)),
            pl.BlockSpec((_K, _D), lambda t: (0, 0)),
            pl.BlockSpec((_TB, 1), lambda t: (t, 0)),
            pl.BlockSpec((1, _K), lambda t: (0, 0)),
        ],
        out_specs=pl.BlockSpec((_TB, 1), lambda t: (t, 0)),
        out_shape=jax.ShapeDtypeStruct((_T, 1), jnp.int32),
        compiler_params=pltpu.CompilerParams(
            dimension_semantics=("arbitrary",)),
    )(x2d, cb, xn, cn)
    return idx2.reshape(_T)


_CH = 128


def _make_gather():
    info = plsc.get_sparse_core_info()
    nw = info.num_cores * info.num_subcores
    b_per_w = _T // nw
    n_chunks = b_per_w // _CH
    mesh = plsc.VectorSubcoreMesh(core_axis_name="c", subcore_axis_name="s")

    @functools.partial(
        pl.kernel, mesh=mesh,
        out_type=jax.ShapeDtypeStruct((_T, _D), jnp.float32),
        scratch_types=[
            pltpu.VMEM((_CH,), jnp.int32),
            pltpu.VMEM((_CH, _D), jnp.float32),
            pltpu.SemaphoreType.DMA,
        ],
    )
    def gather(idx_hbm, table_hbm, out_hbm, idx_v, rows_v, sem):
        wid = lax.axis_index("s") * info.num_cores + lax.axis_index("c")
        base = wid * b_per_w
        for c in range(n_chunks):
            off = base + c * _CH
            pltpu.sync_copy(idx_hbm.at[pl.ds(off, _CH)], idx_v)
            pltpu.async_copy(table_hbm.at[idx_v], rows_v, sem).wait()
            pltpu.sync_copy(rows_v, out_hbm.at[pl.ds(off, _CH)])

    return gather


def kernel(x, codebook):
    B, S, D = x.shape
    x2d = x.reshape(B * S, D)
    cb = codebook.reshape(-1, D)
    xn = jnp.sum(x ** 2, axis=-1, keepdims=True).reshape(B * S, 1)
    cn = jnp.sum(codebook[0] ** 2, axis=-1).reshape(1, -1)
    idx = _argmin_call(x2d, cb, xn, cn)
    out = _make_gather()(idx, cb)
    return out.reshape(B, S, D)


# speedup vs baseline: 1.3831x; 1.1220x over previous
import functools

import jax
import jax.numpy as jnp
from jax import lax
from jax.experimental import pallas as pl
from jax.experimental.pallas import tpu as pltpu
from jax.experimental.pallas import tpu_sc as plsc

_T = 16 * 1024
_D = 256
_K = 8192

_TB = 128
_MM = 1024
_LW = 128
_NT = _T // _TB
_WINDOW_END_STEPS = (2816 // _LW - 1, 5632 // _LW - 1, _K // _LW - 1)


_STEPS_PER_MM = _MM // _LW


def _plan_groups():
    groups, prev = [], -1
    for end in _WINDOW_END_STEPS:
        g = prev + 1
        while g <= end:
            size = min(4, end - g + 1, _STEPS_PER_MM - (g % _STEPS_PER_MM))
            groups.append((g, size, g + size - 1 == end))
            g += size
        prev = end
    return groups

_GROUPS = _plan_groups()


def _argmin_body(x_ref, cb2_ref, xn_ref, cn_ref, idx_ref):
    x_bf = x_ref[...]
    xn = xn_ref[...]

    inf1 = jnp.full((_TB, 1), jnp.inf, jnp.float32)
    infl = jnp.full((_TB, _LW), jnp.inf, jnp.float32)
    zerl = jnp.zeros((_TB, _LW), jnp.float32)
    lane_iota = lax.broadcasted_iota(
        jnp.int32, (_TB, _LW), 1).astype(jnp.float32)
    lane_v, lane_s = infl, zerl
    acc_v, acc_i = inf1, jnp.zeros((_TB, 1), jnp.float32)

    s2 = None
    cur_mm = -1
    for gstep0, size, win_end in _GROUPS:
        if gstep0 // _STEPS_PER_MM != cur_mm:
            cur_mm = gstep0 // _STEPS_PER_MM
            cb2_blk = cb2_ref[pl.ds(cur_mm * _MM, _MM), :]
            s2 = lax.dot_general(
                x_bf, cb2_blk, (((1,), (1,)), ((), ())),
                preferred_element_type=jnp.float32)

        def dstep(g):
            j = g % _STEPS_PER_MM
            cn_blk = cn_ref[0:1, pl.ds(g * _LW, _LW)]
            return (xn + cn_blk) - s2[:, j * _LW:(j + 1) * _LW]

        def pair(d0, s0, d1, s1):
            v = jnp.minimum(d0, d1)
            lt = d1 < d0
            s_ = jnp.where(lt, s1, s0)
            return v, s_

        g = gstep0
        if size == 4:
            d0, d1, d2, d3 = dstep(g), dstep(g + 1), dstep(g + 2), dstep(g + 3)
            v01, s01 = pair(d0, jnp.float32(g), d1, jnp.float32(g + 1))
            v23, s23 = pair(d2, jnp.float32(g + 2), d3, jnp.float32(g + 3))
            v_g = jnp.minimum(v01, v23)
            s_g = jnp.where(v23 < v01, s23, s01)
        elif size == 2:
            v_g, s_g = pair(dstep(g), jnp.float32(g),
                            dstep(g + 1), jnp.float32(g + 1))
        else:
            v_g, s_g = dstep(g), jnp.full((_TB, _LW), jnp.float32(g))

        upd = v_g < lane_v
        lane_v = jnp.where(upd, v_g, lane_v)
        lane_s = jnp.where(upd, s_g, lane_s)

        if win_end:
            m = jnp.min(lane_v, axis=1, keepdims=True)
            kk = lane_s * jnp.float32(_LW) + lane_iota
            io = jnp.min(jnp.where(lane_v == m, kk, jnp.float32(1e9)),
                         axis=1, keepdims=True)
            lt = m < acc_v
            acc_i = jnp.where(lt, io, acc_i)
            acc_v = jnp.where(lt, m, acc_v)
            acc_v = acc_v.astype(jnp.bfloat16).astype(jnp.float32)
            lane_v, lane_s = infl, zerl

    idx_ref[...] = acc_i.astype(jnp.int32)


def _argmin_call(x_bf, cb2_bf, xn, cn):
    idx2 = pl.pallas_call(
        _argmin_body,
        grid=(_NT,),
        in_specs=[
            pl.BlockSpec((_TB, _D), lambda t: (t, 0)),
            pl.BlockSpec((_K, _D), lambda t: (0, 0)),
            pl.BlockSpec((_TB, 1), lambda t: (t, 0)),
            pl.BlockSpec((1, _K), lambda t: (0, 0)),
        ],
        out_specs=pl.BlockSpec((_TB, 1), lambda t: (t, 0)),
        out_shape=jax.ShapeDtypeStruct((_T, 1), jnp.int32),
        compiler_params=pltpu.CompilerParams(
            dimension_semantics=("arbitrary",)),
    )(x_bf, cb2_bf, xn, cn)
    return idx2.reshape(_T)


_CH = 128


def _make_gather():
    info = plsc.get_sparse_core_info()
    nw = info.num_cores * info.num_subcores
    b_per_w = _T // nw
    n_chunks = b_per_w // _CH
    mesh = plsc.VectorSubcoreMesh(core_axis_name="c", subcore_axis_name="s")

    @functools.partial(
        pl.kernel, mesh=mesh,
        out_type=jax.ShapeDtypeStruct((_T, _D), jnp.float32),
        scratch_types=[
            pltpu.VMEM((_CH,), jnp.int32),
            pltpu.VMEM((_CH, _D), jnp.float32),
            pltpu.SemaphoreType.DMA,
        ],
    )
    def gather(idx_hbm, table_hbm, out_hbm, idx_v, rows_v, sem):
        wid = lax.axis_index("s") * info.num_cores + lax.axis_index("c")
        base = wid * b_per_w
        for c in range(n_chunks):
            off = base + c * _CH
            pltpu.sync_copy(idx_hbm.at[pl.ds(off, _CH)], idx_v)
            pltpu.async_copy(table_hbm.at[idx_v], rows_v, sem).wait()
            pltpu.sync_copy(rows_v, out_hbm.at[pl.ds(off, _CH)])

    return gather


def kernel(x, codebook):
    B, S, D = x.shape
    x2d = x.reshape(B * S, D)
    cb = codebook.reshape(-1, D)
    xn = jnp.sum(x ** 2, axis=-1, keepdims=True).reshape(B * S, 1)
    cn = jnp.sum(codebook[0] ** 2, axis=-1).reshape(1, -1)
    x_bf = x2d.astype(jnp.bfloat16)
    cb2_bf = (cb * 2.0).astype(jnp.bfloat16)
    idx = _argmin_call(x_bf, cb2_bf, xn, cn)
    out = _make_gather()(idx, cb)
    return out.reshape(B, S, D)


# speedup vs baseline: 2.0236x; 1.4632x over previous
import functools

import jax
import jax.numpy as jnp
from jax import lax
from jax.experimental import pallas as pl
from jax.experimental.pallas import tpu as pltpu
from jax.experimental.pallas import tpu_sc as plsc

_T = 16 * 1024
_D = 256
_K = 8192

_TB = 256
_MM = 1024
_LW = 128
_NT = _T // _TB
_WINDOW_END_STEPS = (2816 // _LW - 1, 5632 // _LW - 1, _K // _LW - 1)


_STEPS_PER_MM = _MM // _LW


def _plan_groups():
    groups, prev = [], -1
    for end in _WINDOW_END_STEPS:
        g = prev + 1
        while g <= end:
            size = min(4, end - g + 1, _STEPS_PER_MM - (g % _STEPS_PER_MM))
            groups.append((g, size, g + size - 1 == end))
            g += size
        prev = end
    return groups

_GROUPS = _plan_groups()


def _argmin_body(x_ref, cb2_ref, xn_ref, cn_ref, idx_ref):
    x_bf = x_ref[...]
    xn = xn_ref[...]

    inf1 = jnp.full((_TB, 1), jnp.inf, jnp.float32)
    infl = jnp.full((_TB, _LW), jnp.inf, jnp.float32)
    zerl = jnp.zeros((_TB, _LW), jnp.float32)
    lane_iota = lax.broadcasted_iota(
        jnp.int32, (_TB, _LW), 1).astype(jnp.float32)
    lane_v, lane_s = infl, zerl
    acc_v, acc_i = inf1, jnp.zeros((_TB, 1), jnp.float32)

    s2 = None
    cur_mm = -1
    for gstep0, size, win_end in _GROUPS:
        if gstep0 // _STEPS_PER_MM != cur_mm:
            cur_mm = gstep0 // _STEPS_PER_MM
            cb2_blk = cb2_ref[pl.ds(cur_mm * _MM, _MM), :]
            s2 = lax.dot_general(
                x_bf, cb2_blk, (((1,), (1,)), ((), ())),
                preferred_element_type=jnp.float32)

        def dstep(g):
            j = g % _STEPS_PER_MM
            cn_blk = cn_ref[0:1, pl.ds(g * _LW, _LW)]
            return (xn + cn_blk) - s2[:, j * _LW:(j + 1) * _LW]

        def pair(d0, s0, d1, s1):
            v = jnp.minimum(d0, d1)
            lt = d1 < d0
            s_ = jnp.where(lt, s1, s0)
            return v, s_

        g = gstep0
        if size == 4:
            d0, d1, d2, d3 = dstep(g), dstep(g + 1), dstep(g + 2), dstep(g + 3)
            v01, s01 = pair(d0, jnp.float32(g), d1, jnp.float32(g + 1))
            v23, s23 = pair(d2, jnp.float32(g + 2), d3, jnp.float32(g + 3))
            v_g = jnp.minimum(v01, v23)
            s_g = jnp.where(v23 < v01, s23, s01)
        elif size == 2:
            v_g, s_g = pair(dstep(g), jnp.float32(g),
                            dstep(g + 1), jnp.float32(g + 1))
        else:
            v_g, s_g = dstep(g), jnp.full((_TB, _LW), jnp.float32(g))

        upd = v_g < lane_v
        lane_v = jnp.where(upd, v_g, lane_v)
        lane_s = jnp.where(upd, s_g, lane_s)

        if win_end:
            m = jnp.min(lane_v, axis=1, keepdims=True)
            kk = lane_s * jnp.float32(_LW) + lane_iota
            io = jnp.min(jnp.where(lane_v == m, kk, jnp.float32(1e9)),
                         axis=1, keepdims=True)
            lt = m < acc_v
            acc_i = jnp.where(lt, io, acc_i)
            acc_v = jnp.where(lt, m, acc_v)
            acc_v = acc_v.astype(jnp.bfloat16).astype(jnp.float32)
            lane_v, lane_s = infl, zerl

    idx_ref[...] = acc_i.astype(jnp.int32)


def _argmin_call(x_bf, cb2_bf, xn, cn):
    idx2 = pl.pallas_call(
        _argmin_body,
        grid=(_NT,),
        in_specs=[
            pl.BlockSpec((_TB, _D), lambda t: (t, 0)),
            pl.BlockSpec((_K, _D), lambda t: (0, 0)),
            pl.BlockSpec((_TB, 1), lambda t: (t, 0)),
            pl.BlockSpec((1, _K), lambda t: (0, 0)),
        ],
        out_specs=pl.BlockSpec((_TB, 1), lambda t: (t, 0)),
        out_shape=jax.ShapeDtypeStruct((_T, 1), jnp.int32),
        compiler_params=pltpu.CompilerParams(
            dimension_semantics=("arbitrary",)),
    )(x_bf, cb2_bf, xn, cn)
    return idx2.reshape(_T)


_CH = 128


def _make_gather():
    info = plsc.get_sparse_core_info()
    nw = info.num_cores * info.num_subcores
    b_per_w = _T // nw
    n_chunks = b_per_w // _CH
    mesh = plsc.VectorSubcoreMesh(core_axis_name="c", subcore_axis_name="s")

    @functools.partial(
        pl.kernel, mesh=mesh,
        out_type=jax.ShapeDtypeStruct((_T, _D), jnp.float32),
        scratch_types=[
            pltpu.VMEM((_CH,), jnp.int32),
            pltpu.VMEM((_CH, _D), jnp.float32),
            pltpu.SemaphoreType.DMA,
        ],
    )
    def gather(idx_hbm, table_hbm, out_hbm, idx_v, rows_v, sem):
        wid = lax.axis_index("s") * info.num_cores + lax.axis_index("c")
        base = wid * b_per_w
        for c in range(n_chunks):
            off = base + c * _CH
            pltpu.sync_copy(idx_hbm.at[pl.ds(off, _CH)], idx_v)
            pltpu.async_copy(table_hbm.at[idx_v], rows_v, sem).wait()
            pltpu.sync_copy(rows_v, out_hbm.at[pl.ds(off, _CH)])

    return gather


def kernel(x, codebook):
    B, S, D = x.shape
    x2d = x.reshape(B * S, D)
    cb = codebook.reshape(-1, D)
    xn = jnp.sum(x ** 2, axis=-1, keepdims=True).reshape(B * S, 1)
    cn = jnp.sum(codebook[0] ** 2, axis=-1).reshape(1, -1)
    x_bf = x2d.astype(jnp.bfloat16)
    cb2_bf = (cb * 2.0).astype(jnp.bfloat16)
    idx = _argmin_call(x_bf, cb2_bf, xn, cn)
    out = _make_gather()(idx, cb)
    return out.reshape(B, S, D)


# speedup vs baseline: 2.2209x; 1.0975x over previous
import functools

import jax
import jax.numpy as jnp
from jax import lax
from jax.experimental import pallas as pl
from jax.experimental.pallas import tpu as pltpu
from jax.experimental.pallas import tpu_sc as plsc

_T = 16 * 1024
_D = 256
_K = 8192

_TB = 512
_MM = 1024
_LW = 128
_NT = _T // _TB
_WINDOW_END_STEPS = (2816 // _LW - 1, 5632 // _LW - 1, _K // _LW - 1)


_STEPS_PER_MM = _MM // _LW


def _plan_groups():
    groups, prev = [], -1
    for end in _WINDOW_END_STEPS:
        g = prev + 1
        while g <= end:
            size = min(4, end - g + 1, _STEPS_PER_MM - (g % _STEPS_PER_MM))
            groups.append((g, size, g + size - 1 == end))
            g += size
        prev = end
    return groups

_GROUPS = _plan_groups()


def _argmin_body(x_ref, cb2_ref, xn_ref, cn_ref, idx_ref):
    x_bf = x_ref[...]
    xn = xn_ref[...]

    inf1 = jnp.full((_TB, 1), jnp.inf, jnp.float32)
    infl = jnp.full((_TB, _LW), jnp.inf, jnp.float32)
    zerl = jnp.zeros((_TB, _LW), jnp.float32)
    lane_iota = lax.broadcasted_iota(
        jnp.int32, (_TB, _LW), 1).astype(jnp.float32)
    lane_v, lane_s = infl, zerl
    acc_v, acc_i = inf1, jnp.zeros((_TB, 1), jnp.float32)

    s2 = None
    cur_mm = -1
    for gstep0, size, win_end in _GROUPS:
        if gstep0 // _STEPS_PER_MM != cur_mm:
            cur_mm = gstep0 // _STEPS_PER_MM
            cb2_blk = cb2_ref[pl.ds(cur_mm * _MM, _MM), :]
            s2 = lax.dot_general(
                x_bf, cb2_blk, (((1,), (1,)), ((), ())),
                preferred_element_type=jnp.float32)

        def dstep(g):
            j = g % _STEPS_PER_MM
            cn_blk = cn_ref[0:1, pl.ds(g * _LW, _LW)]
            return (xn + cn_blk) - s2[:, j * _LW:(j + 1) * _LW]

        def pair(d0, s0, d1, s1):
            v = jnp.minimum(d0, d1)
            lt = d1 < d0
            s_ = jnp.where(lt, s1, s0)
            return v, s_

        g = gstep0
        if size == 4:
            d0, d1, d2, d3 = dstep(g), dstep(g + 1), dstep(g + 2), dstep(g + 3)
            v01, s01 = pair(d0, jnp.float32(g), d1, jnp.float32(g + 1))
            v23, s23 = pair(d2, jnp.float32(g + 2), d3, jnp.float32(g + 3))
            v_g = jnp.minimum(v01, v23)
            s_g = jnp.where(v23 < v01, s23, s01)
        elif size == 2:
            v_g, s_g = pair(dstep(g), jnp.float32(g),
                            dstep(g + 1), jnp.float32(g + 1))
        else:
            v_g, s_g = dstep(g), jnp.full((_TB, _LW), jnp.float32(g))

        upd = v_g < lane_v
        lane_v = jnp.where(upd, v_g, lane_v)
        lane_s = jnp.where(upd, s_g, lane_s)

        if win_end:
            m = jnp.min(lane_v, axis=1, keepdims=True)
            kk = lane_s * jnp.float32(_LW) + lane_iota
            io = jnp.min(jnp.where(lane_v == m, kk, jnp.float32(1e9)),
                         axis=1, keepdims=True)
            lt = m < acc_v
            acc_i = jnp.where(lt, io, acc_i)
            acc_v = jnp.where(lt, m, acc_v)
            acc_v = acc_v.astype(jnp.bfloat16).astype(jnp.float32)
            lane_v, lane_s = infl, zerl

    idx_ref[...] = acc_i.astype(jnp.int32)


def _argmin_call(x_bf, cb2_bf, xn, cn):
    idx2 = pl.pallas_call(
        _argmin_body,
        grid=(_NT,),
        in_specs=[
            pl.BlockSpec((_TB, _D), lambda t: (t, 0)),
            pl.BlockSpec((_K, _D), lambda t: (0, 0)),
            pl.BlockSpec((_TB, 1), lambda t: (t, 0)),
            pl.BlockSpec((1, _K), lambda t: (0, 0)),
        ],
        out_specs=pl.BlockSpec((_TB, 1), lambda t: (t, 0)),
        out_shape=jax.ShapeDtypeStruct((_T, 1), jnp.int32),
        compiler_params=pltpu.CompilerParams(
            dimension_semantics=("arbitrary",)),
    )(x_bf, cb2_bf, xn, cn)
    return idx2.reshape(_T)


_CH = 128


def _make_gather():
    info = plsc.get_sparse_core_info()
    nw = info.num_cores * info.num_subcores
    b_per_w = _T // nw
    n_chunks = b_per_w // _CH
    mesh = plsc.VectorSubcoreMesh(core_axis_name="c", subcore_axis_name="s")

    @functools.partial(
        pl.kernel, mesh=mesh,
        out_type=jax.ShapeDtypeStruct((_T, _D), jnp.float32),
        scratch_types=[
            pltpu.VMEM((_CH,), jnp.int32),
            pltpu.VMEM((_CH, _D), jnp.float32),
            pltpu.SemaphoreType.DMA,
        ],
    )
    def gather(idx_hbm, table_hbm, out_hbm, idx_v, rows_v, sem):
        wid = lax.axis_index("s") * info.num_cores + lax.axis_index("c")
        base = wid * b_per_w
        for c in range(n_chunks):
            off = base + c * _CH
            pltpu.sync_copy(idx_hbm.at[pl.ds(off, _CH)], idx_v)
            pltpu.async_copy(table_hbm.at[idx_v], rows_v, sem).wait()
            pltpu.sync_copy(rows_v, out_hbm.at[pl.ds(off, _CH)])

    return gather


def kernel(x, codebook):
    B, S, D = x.shape
    x2d = x.reshape(B * S, D)
    cb = codebook.reshape(-1, D)
    xn = jnp.sum(x ** 2, axis=-1, keepdims=True).reshape(B * S, 1)
    cn = jnp.sum(codebook[0] ** 2, axis=-1).reshape(1, -1)
    x_bf = x2d.astype(jnp.bfloat16)
    cb2_bf = (cb * 2.0).astype(jnp.bfloat16)
    idx = _argmin_call(x_bf, cb2_bf, xn, cn)
    out = _make_gather()(idx, cb)
    return out.reshape(B, S, D)


# speedup vs baseline: 2.2563x; 1.0159x over previous
import functools

import jax
import jax.numpy as jnp
from jax import lax
from jax.experimental import pallas as pl
from jax.experimental.pallas import tpu as pltpu
from jax.experimental.pallas import tpu_sc as plsc

_T = 16 * 1024
_D = 256
_K = 8192

_TB = 1024
_MM = 1024
_LW = 128
_NT = _T // _TB
_WINDOW_END_STEPS = (2816 // _LW - 1, 5632 // _LW - 1, _K // _LW - 1)


_STEPS_PER_MM = _MM // _LW


def _plan_groups():
    groups, prev = [], -1
    for end in _WINDOW_END_STEPS:
        g = prev + 1
        while g <= end:
            size = min(4, end - g + 1, _STEPS_PER_MM - (g % _STEPS_PER_MM))
            groups.append((g, size, g + size - 1 == end))
            g += size
        prev = end
    return groups

_GROUPS = _plan_groups()


def _argmin_body(x_ref, cb2_ref, xn_ref, cn_ref, idx_ref):
    x_bf = x_ref[...]
    xn = xn_ref[...]

    inf1 = jnp.full((_TB, 1), jnp.inf, jnp.float32)
    infl = jnp.full((_TB, _LW), jnp.inf, jnp.float32)
    zerl = jnp.zeros((_TB, _LW), jnp.float32)
    lane_iota = lax.broadcasted_iota(
        jnp.int32, (_TB, _LW), 1).astype(jnp.float32)
    lane_v, lane_s = infl, zerl
    acc_v, acc_i = inf1, jnp.zeros((_TB, 1), jnp.float32)

    s2 = None
    cur_mm = -1
    for gstep0, size, win_end in _GROUPS:
        if gstep0 // _STEPS_PER_MM != cur_mm:
            cur_mm = gstep0 // _STEPS_PER_MM
            cb2_blk = cb2_ref[pl.ds(cur_mm * _MM, _MM), :]
            s2 = lax.dot_general(
                x_bf, cb2_blk, (((1,), (1,)), ((), ())),
                preferred_element_type=jnp.float32)

        def dstep(g):
            j = g % _STEPS_PER_MM
            cn_blk = cn_ref[0:1, pl.ds(g * _LW, _LW)]
            return (xn + cn_blk) - s2[:, j * _LW:(j + 1) * _LW]

        def pair(d0, s0, d1, s1):
            v = jnp.minimum(d0, d1)
            lt = d1 < d0
            s_ = jnp.where(lt, s1, s0)
            return v, s_

        g = gstep0
        if size == 4:
            d0, d1, d2, d3 = dstep(g), dstep(g + 1), dstep(g + 2), dstep(g + 3)
            v01, s01 = pair(d0, jnp.float32(g), d1, jnp.float32(g + 1))
            v23, s23 = pair(d2, jnp.float32(g + 2), d3, jnp.float32(g + 3))
            v_g = jnp.minimum(v01, v23)
            s_g = jnp.where(v23 < v01, s23, s01)
        elif size == 2:
            v_g, s_g = pair(dstep(g), jnp.float32(g),
                            dstep(g + 1), jnp.float32(g + 1))
        else:
            v_g, s_g = dstep(g), jnp.full((_TB, _LW), jnp.float32(g))

        upd = v_g < lane_v
        lane_v = jnp.where(upd, v_g, lane_v)
        lane_s = jnp.where(upd, s_g, lane_s)

        if win_end:
            m = jnp.min(lane_v, axis=1, keepdims=True)
            kk = lane_s * jnp.float32(_LW) + lane_iota
            io = jnp.min(jnp.where(lane_v == m, kk, jnp.float32(1e9)),
                         axis=1, keepdims=True)
            lt = m < acc_v
            acc_i = jnp.where(lt, io, acc_i)
            acc_v = jnp.where(lt, m, acc_v)
            acc_v = acc_v.astype(jnp.bfloat16).astype(jnp.float32)
            lane_v, lane_s = infl, zerl

    idx_ref[...] = acc_i.astype(jnp.int32)


def _argmin_call(x_bf, cb2_bf, xn, cn):
    idx2 = pl.pallas_call(
        _argmin_body,
        grid=(_NT,),
        in_specs=[
            pl.BlockSpec((_TB, _D), lambda t: (t, 0)),
            pl.BlockSpec((_K, _D), lambda t: (0, 0)),
            pl.BlockSpec((_TB, 1), lambda t: (t, 0)),
            pl.BlockSpec((1, _K), lambda t: (0, 0)),
        ],
        out_specs=pl.BlockSpec((_TB, 1), lambda t: (t, 0)),
        out_shape=jax.ShapeDtypeStruct((_T, 1), jnp.int32),
        compiler_params=pltpu.CompilerParams(
            dimension_semantics=("arbitrary",)),
    )(x_bf, cb2_bf, xn, cn)
    return idx2.reshape(_T)


_CH = 128


def _make_gather():
    info = plsc.get_sparse_core_info()
    nw = info.num_cores * info.num_subcores
    b_per_w = _T // nw
    n_chunks = b_per_w // _CH
    mesh = plsc.VectorSubcoreMesh(core_axis_name="c", subcore_axis_name="s")

    @functools.partial(
        pl.kernel, mesh=mesh,
        out_type=jax.ShapeDtypeStruct((_T, _D), jnp.float32),
        scratch_types=[
            pltpu.VMEM((_CH,), jnp.int32),
            pltpu.VMEM((_CH, _D), jnp.float32),
            pltpu.SemaphoreType.DMA,
        ],
    )
    def gather(idx_hbm, table_hbm, out_hbm, idx_v, rows_v, sem):
        wid = lax.axis_index("s") * info.num_cores + lax.axis_index("c")
        base = wid * b_per_w
        for c in range(n_chunks):
            off = base + c * _CH
            pltpu.sync_copy(idx_hbm.at[pl.ds(off, _CH)], idx_v)
            pltpu.async_copy(table_hbm.at[idx_v], rows_v, sem).wait()
            pltpu.sync_copy(rows_v, out_hbm.at[pl.ds(off, _CH)])

    return gather


def kernel(x, codebook):
    B, S, D = x.shape
    x2d = x.reshape(B * S, D)
    cb = codebook.reshape(-1, D)
    xn = jnp.sum(x ** 2, axis=-1, keepdims=True).reshape(B * S, 1)
    cn = jnp.sum(codebook[0] ** 2, axis=-1).reshape(1, -1)
    x_bf = x2d.astype(jnp.bfloat16)
    cb2_bf = (cb * 2.0).astype(jnp.bfloat16)
    idx = _argmin_call(x_bf, cb2_bf, xn, cn)
    out = _make_gather()(idx, cb)
    return out.reshape(B, S, D)


# speedup vs baseline: 2.3009x; 1.0198x over previous
import functools

import jax
import jax.numpy as jnp
from jax import lax
from jax.experimental import pallas as pl
from jax.experimental.pallas import tpu as pltpu
from jax.experimental.pallas import tpu_sc as plsc

_T = 16 * 1024
_D = 256
_K = 8192

_TB = 2048
_MM = 1024
_LW = 128
_NT = _T // _TB
_WINDOW_END_STEPS = (2816 // _LW - 1, 5632 // _LW - 1, _K // _LW - 1)


_STEPS_PER_MM = _MM // _LW


def _plan_groups():
    groups, prev = [], -1
    for end in _WINDOW_END_STEPS:
        g = prev + 1
        while g <= end:
            size = min(4, end - g + 1, _STEPS_PER_MM - (g % _STEPS_PER_MM))
            groups.append((g, size, g + size - 1 == end))
            g += size
        prev = end
    return groups

_GROUPS = _plan_groups()


def _argmin_body(x_ref, cb2_ref, xn_ref, cn_ref, idx_ref):
    x_bf = x_ref[...]
    xn = xn_ref[...]

    inf1 = jnp.full((_TB, 1), jnp.inf, jnp.float32)
    infl = jnp.full((_TB, _LW), jnp.inf, jnp.float32)
    zerl = jnp.zeros((_TB, _LW), jnp.float32)
    lane_iota = lax.broadcasted_iota(
        jnp.int32, (_TB, _LW), 1).astype(jnp.float32)
    lane_v, lane_s = infl, zerl
    acc_v, acc_i = inf1, jnp.zeros((_TB, 1), jnp.float32)

    s2 = None
    cur_mm = -1
    for gstep0, size, win_end in _GROUPS:
        if gstep0 // _STEPS_PER_MM != cur_mm:
            cur_mm = gstep0 // _STEPS_PER_MM
            cb2_blk = cb2_ref[pl.ds(cur_mm * _MM, _MM), :]
            s2 = lax.dot_general(
                x_bf, cb2_blk, (((1,), (1,)), ((), ())),
                preferred_element_type=jnp.float32)

        def dstep(g):
            j = g % _STEPS_PER_MM
            cn_blk = cn_ref[0:1, pl.ds(g * _LW, _LW)]
            return (xn + cn_blk) - s2[:, j * _LW:(j + 1) * _LW]

        def pair(d0, s0, d1, s1):
            v = jnp.minimum(d0, d1)
            lt = d1 < d0
            s_ = jnp.where(lt, s1, s0)
            return v, s_

        g = gstep0
        if size == 4:
            d0, d1, d2, d3 = dstep(g), dstep(g + 1), dstep(g + 2), dstep(g + 3)
            v01, s01 = pair(d0, jnp.float32(g), d1, jnp.float32(g + 1))
            v23, s23 = pair(d2, jnp.float32(g + 2), d3, jnp.float32(g + 3))
            v_g = jnp.minimum(v01, v23)
            s_g = jnp.where(v23 < v01, s23, s01)
        elif size == 2:
            v_g, s_g = pair(dstep(g), jnp.float32(g),
                            dstep(g + 1), jnp.float32(g + 1))
        else:
            v_g, s_g = dstep(g), jnp.full((_TB, _LW), jnp.float32(g))

        upd = v_g < lane_v
        lane_v = jnp.where(upd, v_g, lane_v)
        lane_s = jnp.where(upd, s_g, lane_s)

        if win_end:
            m = jnp.min(lane_v, axis=1, keepdims=True)
            kk = lane_s * jnp.float32(_LW) + lane_iota
            io = jnp.min(jnp.where(lane_v == m, kk, jnp.float32(1e9)),
                         axis=1, keepdims=True)
            lt = m < acc_v
            acc_i = jnp.where(lt, io, acc_i)
            acc_v = jnp.where(lt, m, acc_v)
            acc_v = acc_v.astype(jnp.bfloat16).astype(jnp.float32)
            lane_v, lane_s = infl, zerl

    idx_ref[...] = acc_i.astype(jnp.int32)


def _argmin_call(x_bf, cb2_bf, xn, cn):
    idx2 = pl.pallas_call(
        _argmin_body,
        grid=(_NT,),
        in_specs=[
            pl.BlockSpec((_TB, _D), lambda t: (t, 0)),
            pl.BlockSpec((_K, _D), lambda t: (0, 0)),
            pl.BlockSpec((_TB, 1), lambda t: (t, 0)),
            pl.BlockSpec((1, _K), lambda t: (0, 0)),
        ],
        out_specs=pl.BlockSpec((_TB, 1), lambda t: (t, 0)),
        out_shape=jax.ShapeDtypeStruct((_T, 1), jnp.int32),
        compiler_params=pltpu.CompilerParams(
            dimension_semantics=("arbitrary",)),
    )(x_bf, cb2_bf, xn, cn)
    return idx2.reshape(_T)


_CH = 128


def _make_gather():
    info = plsc.get_sparse_core_info()
    nw = info.num_cores * info.num_subcores
    b_per_w = _T // nw
    n_chunks = b_per_w // _CH
    mesh = plsc.VectorSubcoreMesh(core_axis_name="c", subcore_axis_name="s")

    @functools.partial(
        pl.kernel, mesh=mesh,
        out_type=jax.ShapeDtypeStruct((_T, _D), jnp.float32),
        scratch_types=[
            pltpu.VMEM((_CH,), jnp.int32),
            pltpu.VMEM((_CH, _D), jnp.float32),
            pltpu.SemaphoreType.DMA,
        ],
    )
    def gather(idx_hbm, table_hbm, out_hbm, idx_v, rows_v, sem):
        wid = lax.axis_index("s") * info.num_cores + lax.axis_index("c")
        base = wid * b_per_w
        for c in range(n_chunks):
            off = base + c * _CH
            pltpu.sync_copy(idx_hbm.at[pl.ds(off, _CH)], idx_v)
            pltpu.async_copy(table_hbm.at[idx_v], rows_v, sem).wait()
            pltpu.sync_copy(rows_v, out_hbm.at[pl.ds(off, _CH)])

    return gather


def kernel(x, codebook):
    B, S, D = x.shape
    x2d = x.reshape(B * S, D)
    cb = codebook.reshape(-1, D)
    xn = jnp.sum(x ** 2, axis=-1, keepdims=True).reshape(B * S, 1)
    cn = jnp.sum(codebook[0] ** 2, axis=-1).reshape(1, -1)
    x_bf = x2d.astype(jnp.bfloat16)
    cb2_bf = (cb * 2.0).astype(jnp.bfloat16)
    idx = _argmin_call(x_bf, cb2_bf, xn, cn)
    out = _make_gather()(idx, cb)
    return out.reshape(B, S, D)


# speedup vs baseline: 2.3223x; 1.0093x over previous
import functools

import jax
import jax.numpy as jnp
from jax import lax
from jax.experimental import pallas as pl
from jax.experimental.pallas import tpu as pltpu
from jax.experimental.pallas import tpu_sc as plsc

_T = 16 * 1024
_D = 256
_K = 8192

_TB = 4096
_MM = 1024
_LW = 128
_NT = _T // _TB
_WINDOW_END_STEPS = (2816 // _LW - 1, 5632 // _LW - 1, _K // _LW - 1)


_STEPS_PER_MM = _MM // _LW


def _plan_groups():
    groups, prev = [], -1
    for end in _WINDOW_END_STEPS:
        g = prev + 1
        while g <= end:
            size = min(4, end - g + 1, _STEPS_PER_MM - (g % _STEPS_PER_MM))
            groups.append((g, size, g + size - 1 == end))
            g += size
        prev = end
    return groups

_GROUPS = _plan_groups()


def _argmin_body(x_ref, cb2_ref, xn_ref, cn_ref, idx_ref):
    x_bf = x_ref[...]
    xn = xn_ref[...]

    inf1 = jnp.full((_TB, 1), jnp.inf, jnp.float32)
    infl = jnp.full((_TB, _LW), jnp.inf, jnp.float32)
    zerl = jnp.zeros((_TB, _LW), jnp.float32)
    lane_iota = lax.broadcasted_iota(
        jnp.int32, (_TB, _LW), 1).astype(jnp.float32)
    lane_v, lane_s = infl, zerl
    acc_v, acc_i = inf1, jnp.zeros((_TB, 1), jnp.float32)

    s2 = None
    cur_mm = -1
    for gstep0, size, win_end in _GROUPS:
        if gstep0 // _STEPS_PER_MM != cur_mm:
            cur_mm = gstep0 // _STEPS_PER_MM
            cb2_blk = cb2_ref[pl.ds(cur_mm * _MM, _MM), :]
            s2 = lax.dot_general(
                x_bf, cb2_blk, (((1,), (1,)), ((), ())),
                preferred_element_type=jnp.float32)

        def dstep(g):
            j = g % _STEPS_PER_MM
            cn_blk = cn_ref[0:1, pl.ds(g * _LW, _LW)]
            return (xn + cn_blk) - s2[:, j * _LW:(j + 1) * _LW]

        def pair(d0, s0, d1, s1):
            v = jnp.minimum(d0, d1)
            lt = d1 < d0
            s_ = jnp.where(lt, s1, s0)
            return v, s_

        g = gstep0
        if size == 4:
            d0, d1, d2, d3 = dstep(g), dstep(g + 1), dstep(g + 2), dstep(g + 3)
            v01, s01 = pair(d0, jnp.float32(g), d1, jnp.float32(g + 1))
            v23, s23 = pair(d2, jnp.float32(g + 2), d3, jnp.float32(g + 3))
            v_g = jnp.minimum(v01, v23)
            s_g = jnp.where(v23 < v01, s23, s01)
        elif size == 2:
            v_g, s_g = pair(dstep(g), jnp.float32(g),
                            dstep(g + 1), jnp.float32(g + 1))
        else:
            v_g, s_g = dstep(g), jnp.full((_TB, _LW), jnp.float32(g))

        upd = v_g < lane_v
        lane_v = jnp.where(upd, v_g, lane_v)
        lane_s = jnp.where(upd, s_g, lane_s)

        if win_end:
            m = jnp.min(lane_v, axis=1, keepdims=True)
            kk = lane_s * jnp.float32(_LW) + lane_iota
            io = jnp.min(jnp.where(lane_v == m, kk, jnp.float32(1e9)),
                         axis=1, keepdims=True)
            lt = m < acc_v
            acc_i = jnp.where(lt, io, acc_i)
            acc_v = jnp.where(lt, m, acc_v)
            acc_v = acc_v.astype(jnp.bfloat16).astype(jnp.float32)
            lane_v, lane_s = infl, zerl

    idx_ref[...] = acc_i.astype(jnp.int32)


def _argmin_call(x_bf, cb2_bf, xn, cn):
    idx2 = pl.pallas_call(
        _argmin_body,
        grid=(_NT,),
        in_specs=[
            pl.BlockSpec((_TB, _D), lambda t: (t, 0)),
            pl.BlockSpec((_K, _D), lambda t: (0, 0)),
            pl.BlockSpec((_TB, 1), lambda t: (t, 0)),
            pl.BlockSpec((1, _K), lambda t: (0, 0)),
        ],
        out_specs=pl.BlockSpec((_TB, 1), lambda t: (t, 0)),
        out_shape=jax.ShapeDtypeStruct((_T, 1), jnp.int32),
        compiler_params=pltpu.CompilerParams(
            dimension_semantics=("arbitrary",)),
    )(x_bf, cb2_bf, xn, cn)
    return idx2.reshape(_T)


_CH = 128


def _make_gather():
    info = plsc.get_sparse_core_info()
    nw = info.num_cores * info.num_subcores
    b_per_w = _T // nw
    n_chunks = b_per_w // _CH
    mesh = plsc.VectorSubcoreMesh(core_axis_name="c", subcore_axis_name="s")

    @functools.partial(
        pl.kernel, mesh=mesh,
        out_type=jax.ShapeDtypeStruct((_T, _D), jnp.float32),
        scratch_types=[
            pltpu.VMEM((_CH,), jnp.int32),
            pltpu.VMEM((_CH, _D), jnp.float32),
            pltpu.SemaphoreType.DMA,
        ],
    )
    def gather(idx_hbm, table_hbm, out_hbm, idx_v, rows_v, sem):
        wid = lax.axis_index("s") * info.num_cores + lax.axis_index("c")
        base = wid * b_per_w
        for c in range(n_chunks):
            off = base + c * _CH
            pltpu.sync_copy(idx_hbm.at[pl.ds(off, _CH)], idx_v)
            pltpu.async_copy(table_hbm.at[idx_v], rows_v, sem).wait()
            pltpu.sync_copy(rows_v, out_hbm.at[pl.ds(off, _CH)])

    return gather


def kernel(x, codebook):
    B, S, D = x.shape
    x2d = x.reshape(B * S, D)
    cb = codebook.reshape(-1, D)
    xn = jnp.sum(x ** 2, axis=-1, keepdims=True).reshape(B * S, 1)
    cn = jnp.sum(codebook[0] ** 2, axis=-1).reshape(1, -1)
    x_bf = x2d.astype(jnp.bfloat16)
    cb2_bf = (cb * 2.0).astype(jnp.bfloat16)
    idx = _argmin_call(x_bf, cb2_bf, xn, cn)
    out = _make_gather()(idx, cb)
    return out.reshape(B, S, D)
